# Initial kernel scaffold; baseline (speedup 1.0000x reference)
import jax
import jax.numpy as jnp
from jax.experimental import pallas as pl


def kernel(value, depth, pos, params):
    raise NotImplementedError("write your pallas kernel here")


# speedup vs baseline: 2.2515x; 2.2515x over previous
import jax
import jax.numpy as jnp
from jax import lax
from jax.experimental import pallas as pl
from jax.experimental.pallas import tpu as pltpu

E = 768
H = 12
DH = 64
W = 64
VOCAB = 16
SD = 3
NEG = -1e9
TPAD = 128


def _embed_kernel(vs_ref, ds_ref, ps_ref, tok_ref, dep_ref, spa_ref, sos_ref,
                  out_ref):
    B, S = vs_ref.shape
    CH = 512

    def gather(idx_row, tab):
        iot = lax.broadcasted_iota(jnp.int32, (TPAD, CH), 0)
        oh = jnp.where(idx_row == iot, 1.0, 0.0)
        return lax.dot_general(oh, tab, (((0,), (0,)), ((), ())),
                               preferred_element_type=jnp.float32)

    for b in range(B):
        for c in range(S // CH):
            r0 = c * CH
            acc = gather(vs_ref[b:b + 1, r0:r0 + CH], tok_ref[...])
            acc += gather(ds_ref[b:b + 1, r0:r0 + CH], dep_ref[...])
            for a in range(SD):
                acc += gather(ps_ref[a, b:b + 1, r0:r0 + CH], spa_ref[a])
            if c == 0:
                rid = lax.broadcasted_iota(jnp.int32, (CH, E), 0)
                acc = jnp.where(rid == 0, sos_ref[...], acc)
            out_ref[b, r0:r0 + CH, :] = acc


def _attn_kernel(x_ref, g_ref, bb_ref, wq_ref, wk_ref, wv_ref, wo_ref,
                 out_ref, qs_ref, ks_ref, vs_ref, xs_ref):
    B, S = x_ref.shape[0], x_ref.shape[1]
    nb = S // W
    scale = DH ** -0.5
    ii = lax.broadcasted_iota(jnp.int32, (nb, nb), 0)
    jj = lax.broadcasted_iota(jnp.int32, (nb, nb), 1)
    bi = lax.broadcasted_iota(jnp.int32, (nb, W, W), 0)
    wi = lax.broadcasted_iota(jnp.int32, (nb, W, W), 1)
    xi = lax.broadcasted_iota(jnp.int32, (nb, W, W), 2)

    def mm(a, w):
        return lax.dot_general(a, w, (((1,), (0,)), ((), ())),
                               preferred_element_type=jnp.float32)

    CH = 512
    nch = S // CH
    cb = CH // W

    for b in range(B):
        for c in range(nch):
            xc = x_ref[b, c * CH:(c + 1) * CH, :]
            m = jnp.mean(xc, axis=-1, keepdims=True)
            var = jnp.mean((xc - m) ** 2, axis=-1, keepdims=True)
            xs_ref[c * CH:(c + 1) * CH, :] = (
                (xc - m) / jnp.sqrt(var + 1e-5) * g_ref[0] + bb_ref[0]
            ).astype(jnp.bfloat16)
        for c in range(nch):
            xbc = xs_ref[c * CH:(c + 1) * CH, :]
            qc = mm(xbc, wq_ref[...]).astype(jnp.bfloat16)
            kc = mm(xbc, wk_ref[...]).astype(jnp.bfloat16)
            vc = mm(xbc, wv_ref[...]).astype(jnp.bfloat16)
            for h in range(H):
                sl = slice(h * DH, (h + 1) * DH)
                bs = slice(c * cb, (c + 1) * cb)
                qs_ref[h, bs] = qc[:, sl].reshape(cb, W, DH)
                ks_ref[h, bs] = kc[:, sl].reshape(cb, W, DH)
                vs_ref[h, bs] = vc[:, sl].reshape(cb, W, DH)

        def head_body(h, carry):
            qb = qs_ref[h]
            kb = ks_ref[h]
            vb = vs_ref[h]
            qm = jnp.mean(qb.astype(jnp.float32), axis=1)
            km = jnp.mean(kb.astype(jnp.float32), axis=1)
            R = lax.dot_general(qm, km, (((1,), (1,)), ((), ())),
                                preferred_element_type=jnp.float32) * scale
            Rm = jnp.where(jj < ii, R, NEG)
            Re = jnp.exp(Rm - jnp.max(Rm, axis=-1, keepdims=True))
            Rs = Re / jnp.sum(Re, axis=-1, keepdims=True)
            Rs = jnp.where(ii > 0, Rs, 0.0).astype(jnp.bfloat16)
            kr = lax.dot_general(Rs, kb, (((1,), (0,)), ((), ())),
                                 preferred_element_type=jnp.float32
                                 ).astype(jnp.bfloat16)
            vr = lax.dot_general(Rs, vb, (((1,), (0,)), ((), ())),
                                 preferred_element_type=jnp.float32
                                 ).astype(jnp.bfloat16)
            dots_r = lax.dot_general(qb, kr, (((2,), (2,)), ((0,), (0,))),
                                     preferred_element_type=jnp.float32
                                     ) * scale
            dots_s = lax.dot_general(qb, kb, (((2,), (2,)), ((0,), (0,))),
                                     preferred_element_type=jnp.float32
                                     ) * scale
            dots_r = jnp.where(bi > 0, dots_r, NEG)
            dots_s = jnp.where(xi <= wi, dots_s, NEG)
            mx = jnp.maximum(jnp.max(dots_r, axis=-1),
                             jnp.max(dots_s, axis=-1))[..., None]
            er = jnp.exp(dots_r - mx)
            es = jnp.exp(dots_s - mx)
            den = (jnp.sum(er, axis=-1) + jnp.sum(es, axis=-1))[..., None]
            erb = (er / den).astype(jnp.bfloat16)
            esb = (es / den).astype(jnp.bfloat16)
            o = lax.dot_general(erb, vr, (((2,), (1,)), ((0,), (0,))),
                                preferred_element_type=jnp.float32)
            o = o + lax.dot_general(esb, vb, (((2,), (1,)), ((0,), (0,))),
                                    preferred_element_type=jnp.float32)
            qs_ref[h] = o.astype(jnp.bfloat16)
            return carry

        lax.fori_loop(0, H, head_body, 0)
        for c in range(nch):
            rs = slice(c * CH, (c + 1) * CH)
            bs = slice(c * cb, (c + 1) * cb)
            oc = jnp.concatenate(
                [qs_ref[h, bs].reshape(CH, DH) for h in range(H)], axis=1)
            out_ref[b, rs, :] = x_ref[b, rs, :] + mm(oc, wo_ref[...])


def _ffn_kernel(x_ref, g_ref, bb_ref, w1_ref, b1_ref, w2_ref, b2_ref,
                out_ref):
    x = x_ref[0]
    m = jnp.mean(x, axis=-1, keepdims=True)
    var = jnp.mean((x - m) ** 2, axis=-1, keepdims=True)
    hn = ((x - m) / jnp.sqrt(var + 1e-5) * g_ref[0] + bb_ref[0]
          ).astype(jnp.bfloat16)
    a = lax.dot_general(hn, w1_ref[...], (((1,), (0,)), ((), ())),
                        preferred_element_type=jnp.float32) + b1_ref[0]
    gl = jax.nn.gelu(a).astype(jnp.bfloat16)
    out = lax.dot_general(gl, w2_ref[...], (((1,), (0,)), ((), ())),
                          preferred_element_type=jnp.float32) + b2_ref[0]
    out_ref[0] = x + out


def _head_kernel(x_ref, hT_ref, out_ref):
    xb = x_ref[0].astype(jnp.bfloat16)
    out_ref[0] = lax.dot_general(xb, hT_ref[...], (((1,), (0,)), ((), ())),
                                 preferred_element_type=jnp.float32)


def _pad_table(t, rows):
    return jnp.pad(t, ((0, rows - t.shape[0]), (0, 0)))


def kernel(value, depth, pos, params):
    B, S = value.shape
    value = value.astype(jnp.int32)
    depth = depth.astype(jnp.int32)
    pos = pos.astype(jnp.int32)
    z = jnp.zeros((B, 1), jnp.int32)
    vs = jnp.concatenate([z, value[:, :-1]], axis=1)
    ds = jnp.concatenate([z, depth[:, :-1]], axis=1)
    ps = jnp.concatenate([jnp.zeros((SD, B, 1), jnp.int32), pos[:, :, :-1]],
                         axis=2)
    tok = _pad_table(params["tok"], TPAD)
    dep = _pad_table(params["dep"], TPAD)
    spa = jnp.stack([_pad_table(params["spa"][a], TPAD) for a in range(SD)])
    sos = params["sos"].reshape(1, E)

    x = pl.pallas_call(
        _embed_kernel,
        out_shape=jax.ShapeDtypeStruct((B, S, E), jnp.float32),
    )(vs, ds, ps, tok, dep, spa, sos)

    bf = jnp.bfloat16
    for lp in params["layers"]:
        x = pl.pallas_call(
            _attn_kernel,
            out_shape=jax.ShapeDtypeStruct((B, S, E), jnp.float32),
            compiler_params=pltpu.CompilerParams(
                vmem_limit_bytes=62 * 1024 * 1024),
            scratch_shapes=[
                pltpu.VMEM((H, S // W, W, DH), jnp.bfloat16),
                pltpu.VMEM((H, S // W, W, DH), jnp.bfloat16),
                pltpu.VMEM((H, S // W, W, DH), jnp.bfloat16),
                pltpu.VMEM((S, E), jnp.bfloat16),
            ],
        )(x, lp["g1"].reshape(1, E), lp["bn1"].reshape(1, E),
          lp["wq"].astype(bf), lp["wk"].astype(bf), lp["wv"].astype(bf),
          lp["wo"].astype(bf))

        FCH = 256
        x = pl.pallas_call(
            _ffn_kernel,
            grid=(B, S // FCH),
            in_specs=[
                pl.BlockSpec((1, FCH, E), lambda b, c: (b, c, 0)),
                pl.BlockSpec((1, E), lambda b, c: (0, 0)),
                pl.BlockSpec((1, E), lambda b, c: (0, 0)),
                pl.BlockSpec((E, 4 * E), lambda b, c: (0, 0)),
                pl.BlockSpec((1, 4 * E), lambda b, c: (0, 0)),
                pl.BlockSpec((4 * E, E), lambda b, c: (0, 0)),
                pl.BlockSpec((1, E), lambda b, c: (0, 0)),
            ],
            out_specs=pl.BlockSpec((1, FCH, E), lambda b, c: (b, c, 0)),
            out_shape=jax.ShapeDtypeStruct((B, S, E), jnp.float32),
        )(x, lp["g2"].reshape(1, E), lp["bn2"].reshape(1, E),
          lp["w1"].astype(bf), lp["b1"].reshape(1, 4 * E),
          lp["w2"].astype(bf), lp["b2"].reshape(1, E))

    hT = jnp.pad(params["head"], ((0, TPAD - (VOCAB + 1)), (0, 0))
                 ).T.astype(bf)
    logits = pl.pallas_call(
        _head_kernel,
        grid=(B,),
        in_specs=[
            pl.BlockSpec((1, S, E), lambda b: (b, 0, 0)),
            pl.BlockSpec((E, TPAD), lambda b: (0, 0)),
        ],
        out_specs=pl.BlockSpec((1, S, TPAD), lambda b: (b, 0, 0)),
        out_shape=jax.ShapeDtypeStruct((B, S, TPAD), jnp.float32),
    )(x, hT)
    return logits[..., :VOCAB + 1]


# speedup vs baseline: 2.4263x; 1.0776x over previous
import jax
import jax.numpy as jnp
from jax import lax
from jax.experimental import pallas as pl
from jax.experimental.pallas import tpu as pltpu

E = 768
H = 12
DH = 64
W = 64
VOCAB = 16
SD = 3
NEG = -1e9
TPAD = 128


def _embed_kernel(vs_ref, ds_ref, ps_ref, tok_ref, dep_ref, spa_ref, sos_ref,
                  out_ref):
    B, S = vs_ref.shape
    CH = 512

    def gather(idx_row, tab):
        iot = lax.broadcasted_iota(jnp.int32, (TPAD, CH), 0)
        oh = jnp.where(idx_row == iot, 1.0, 0.0)
        return lax.dot_general(oh, tab, (((0,), (0,)), ((), ())),
                               preferred_element_type=jnp.float32)

    for b in range(B):
        for c in range(S // CH):
            r0 = c * CH
            acc = gather(vs_ref[b:b + 1, r0:r0 + CH], tok_ref[...])
            acc += gather(ds_ref[b:b + 1, r0:r0 + CH], dep_ref[...])
            for a in range(SD):
                acc += gather(ps_ref[a, b:b + 1, r0:r0 + CH], spa_ref[a])
            if c == 0:
                rid = lax.broadcasted_iota(jnp.int32, (CH, E), 0)
                acc = jnp.where(rid == 0, sos_ref[...], acc)
            out_ref[b, r0:r0 + CH, :] = acc


def _attn_kernel(x_ref, g_ref, bb_ref, wq_ref, wk_ref, wv_ref, wo_ref,
                 out_ref, qs_ref, ks_ref, vs_ref, xs_ref):
    B, S = x_ref.shape[0], x_ref.shape[1]
    nb = S // W
    scale = DH ** -0.5
    ii = lax.broadcasted_iota(jnp.int32, (nb, nb), 0)
    jj = lax.broadcasted_iota(jnp.int32, (nb, nb), 1)
    bi = lax.broadcasted_iota(jnp.int32, (nb, W, W), 0)
    wi = lax.broadcasted_iota(jnp.int32, (nb, W, W), 1)
    xi = lax.broadcasted_iota(jnp.int32, (nb, W, W), 2)
    amask = jnp.concatenate(
        [jnp.where(bi > 0, 0.0, NEG), jnp.where(xi <= wi, 0.0, NEG)], axis=2)

    def mm(a, w):
        return lax.dot_general(a, w, (((1,), (0,)), ((), ())),
                               preferred_element_type=jnp.float32)

    CH = 512
    nch = S // CH
    cb = CH // W

    for b in range(B):
        for c in range(nch):
            xc = x_ref[b, c * CH:(c + 1) * CH, :]
            m = jnp.mean(xc, axis=-1, keepdims=True)
            var = jnp.mean((xc - m) ** 2, axis=-1, keepdims=True)
            xs_ref[c * CH:(c + 1) * CH, :] = (
                (xc - m) / jnp.sqrt(var + 1e-5) * g_ref[0] + bb_ref[0]
            ).astype(jnp.bfloat16)
        for c in range(nch):
            xbc = xs_ref[c * CH:(c + 1) * CH, :]
            qc = (mm(xbc, wq_ref[...]) * scale).astype(jnp.bfloat16)
            kc = mm(xbc, wk_ref[...]).astype(jnp.bfloat16)
            vc = mm(xbc, wv_ref[...]).astype(jnp.bfloat16)
            for h in range(H):
                sl = slice(h * DH, (h + 1) * DH)
                bs = slice(c * cb, (c + 1) * cb)
                qs_ref[h, bs] = qc[:, sl].reshape(cb, W, DH)
                ks_ref[h, bs] = kc[:, sl].reshape(cb, W, DH)
                vs_ref[h, bs] = vc[:, sl].reshape(cb, W, DH)

        def head_body(h, carry):
            qb = qs_ref[h]
            kb = ks_ref[h]
            vb = vs_ref[h]
            qm = jnp.mean(qb.astype(jnp.float32), axis=1)
            km = jnp.mean(kb.astype(jnp.float32), axis=1)
            R = lax.dot_general(qm, km, (((1,), (1,)), ((), ())),
                                preferred_element_type=jnp.float32)
            Rm = jnp.where(jj < ii, R, NEG)
            Re = jnp.exp(Rm - jnp.max(Rm, axis=-1, keepdims=True))
            Rs = Re / jnp.sum(Re, axis=-1, keepdims=True)
            Rs = jnp.where(ii > 0, Rs, 0.0).astype(jnp.bfloat16)
            kr = lax.dot_general(Rs, kb, (((1,), (0,)), ((), ())),
                                 preferred_element_type=jnp.float32
                                 ).astype(jnp.bfloat16)
            vr = lax.dot_general(Rs, vb, (((1,), (0,)), ((), ())),
                                 preferred_element_type=jnp.float32
                                 ).astype(jnp.bfloat16)
            kcat = jnp.concatenate([kr, kb], axis=1)
            vcat = jnp.concatenate([vr, vb], axis=1)
            dots = lax.dot_general(qb, kcat, (((2,), (2,)), ((0,), (0,))),
                                   preferred_element_type=jnp.float32)
            dots = dots + amask
            mx = jnp.max(dots, axis=-1, keepdims=True)
            e = jnp.exp(dots - mx)
            den = jnp.sum(e, axis=-1, keepdims=True)
            attn = (e / den).astype(jnp.bfloat16)
            o = lax.dot_general(attn, vcat, (((2,), (1,)), ((0,), (0,))),
                                preferred_element_type=jnp.float32)
            qs_ref[h] = o.astype(jnp.bfloat16)
            return carry

        lax.fori_loop(0, H, head_body, 0)
        for c in range(nch):
            rs = slice(c * CH, (c + 1) * CH)
            bs = slice(c * cb, (c + 1) * cb)
            oc = jnp.concatenate(
                [qs_ref[h, bs].reshape(CH, DH) for h in range(H)], axis=1)
            out_ref[b, rs, :] = x_ref[b, rs, :] + mm(oc, wo_ref[...])


def _ffn_kernel(x_ref, g_ref, bb_ref, w1_ref, b1_ref, w2_ref, b2_ref,
                out_ref):
    x = x_ref[0]
    m = jnp.mean(x, axis=-1, keepdims=True)
    var = jnp.mean((x - m) ** 2, axis=-1, keepdims=True)
    hn = ((x - m) / jnp.sqrt(var + 1e-5) * g_ref[0] + bb_ref[0]
          ).astype(jnp.bfloat16)
    a = lax.dot_general(hn, w1_ref[...], (((1,), (0,)), ((), ())),
                        preferred_element_type=jnp.float32) + b1_ref[0]
    gl = jax.nn.gelu(a).astype(jnp.bfloat16)
    out = lax.dot_general(gl, w2_ref[...], (((1,), (0,)), ((), ())),
                          preferred_element_type=jnp.float32) + b2_ref[0]
    out_ref[0] = x + out


def _head_kernel(x_ref, hT_ref, out_ref):
    xb = x_ref[0].astype(jnp.bfloat16)
    out_ref[0] = lax.dot_general(xb, hT_ref[...], (((1,), (0,)), ((), ())),
                                 preferred_element_type=jnp.float32)


def _pad_table(t, rows):
    return jnp.pad(t, ((0, rows - t.shape[0]), (0, 0)))


def kernel(value, depth, pos, params):
    B, S = value.shape
    value = value.astype(jnp.int32)
    depth = depth.astype(jnp.int32)
    pos = pos.astype(jnp.int32)
    z = jnp.zeros((B, 1), jnp.int32)
    vs = jnp.concatenate([z, value[:, :-1]], axis=1)
    ds = jnp.concatenate([z, depth[:, :-1]], axis=1)
    ps = jnp.concatenate([jnp.zeros((SD, B, 1), jnp.int32), pos[:, :, :-1]],
                         axis=2)
    tok = _pad_table(params["tok"], TPAD)
    dep = _pad_table(params["dep"], TPAD)
    spa = jnp.stack([_pad_table(params["spa"][a], TPAD) for a in range(SD)])
    sos = params["sos"].reshape(1, E)

    x = pl.pallas_call(
        _embed_kernel,
        out_shape=jax.ShapeDtypeStruct((B, S, E), jnp.float32),
    )(vs, ds, ps, tok, dep, spa, sos)

    bf = jnp.bfloat16
    for lp in params["layers"]:
        x = pl.pallas_call(
            _attn_kernel,
            out_shape=jax.ShapeDtypeStruct((B, S, E), jnp.float32),
            compiler_params=pltpu.CompilerParams(
                vmem_limit_bytes=62 * 1024 * 1024),
            scratch_shapes=[
                pltpu.VMEM((H, S // W, W, DH), jnp.bfloat16),
                pltpu.VMEM((H, S // W, W, DH), jnp.bfloat16),
                pltpu.VMEM((H, S // W, W, DH), jnp.bfloat16),
                pltpu.VMEM((S, E), jnp.bfloat16),
            ],
        )(x, lp["g1"].reshape(1, E), lp["bn1"].reshape(1, E),
          lp["wq"].astype(bf), lp["wk"].astype(bf), lp["wv"].astype(bf),
          lp["wo"].astype(bf))

        FCH = 256
        x = pl.pallas_call(
            _ffn_kernel,
            grid=(B, S // FCH),
            in_specs=[
                pl.BlockSpec((1, FCH, E), lambda b, c: (b, c, 0)),
                pl.BlockSpec((1, E), lambda b, c: (0, 0)),
                pl.BlockSpec((1, E), lambda b, c: (0, 0)),
                pl.BlockSpec((E, 4 * E), lambda b, c: (0, 0)),
                pl.BlockSpec((1, 4 * E), lambda b, c: (0, 0)),
                pl.BlockSpec((4 * E, E), lambda b, c: (0, 0)),
                pl.BlockSpec((1, E), lambda b, c: (0, 0)),
            ],
            out_specs=pl.BlockSpec((1, FCH, E), lambda b, c: (b, c, 0)),
            out_shape=jax.ShapeDtypeStruct((B, S, E), jnp.float32),
        )(x, lp["g2"].reshape(1, E), lp["bn2"].reshape(1, E),
          lp["w1"].astype(bf), lp["b1"].reshape(1, 4 * E),
          lp["w2"].astype(bf), lp["b2"].reshape(1, E))

    hT = jnp.pad(params["head"], ((0, TPAD - (VOCAB + 1)), (0, 0))
                 ).T.astype(bf)
    logits = pl.pallas_call(
        _head_kernel,
        grid=(B,),
        in_specs=[
            pl.BlockSpec((1, S, E), lambda b: (b, 0, 0)),
            pl.BlockSpec((E, TPAD), lambda b: (0, 0)),
        ],
        out_specs=pl.BlockSpec((1, S, TPAD), lambda b: (b, 0, 0)),
        out_shape=jax.ShapeDtypeStruct((B, S, TPAD), jnp.float32),
    )(x, hT)
    return logits[..., :VOCAB + 1]


# speedup vs baseline: 2.4776x; 1.0211x over previous
import jax
import jax.numpy as jnp
from jax import lax
from jax.experimental import pallas as pl
from jax.experimental.pallas import tpu as pltpu

E = 768
H = 12
DH = 64
W = 64
VOCAB = 16
SD = 3
NEG = -1e9
TPAD = 128


def _embed_kernel(vs_ref, ds_ref, ps_ref, tok_ref, dep_ref, spa_ref, sos_ref,
                  out_ref):
    B, S = vs_ref.shape
    CH = 512

    def gather(idx_row, tab):
        iot = lax.broadcasted_iota(jnp.int32, (TPAD, CH), 0)
        oh = jnp.where(idx_row == iot, 1.0, 0.0)
        return lax.dot_general(oh, tab, (((0,), (0,)), ((), ())),
                               preferred_element_type=jnp.float32)

    for b in range(B):
        for c in range(S // CH):
            r0 = c * CH
            acc = gather(vs_ref[b:b + 1, r0:r0 + CH], tok_ref[...])
            acc += gather(ds_ref[b:b + 1, r0:r0 + CH], dep_ref[...])
            for a in range(SD):
                acc += gather(ps_ref[a, b:b + 1, r0:r0 + CH], spa_ref[a])
            if c == 0:
                rid = lax.broadcasted_iota(jnp.int32, (CH, E), 0)
                acc = jnp.where(rid == 0, sos_ref[...], acc)
            out_ref[b, r0:r0 + CH, :] = acc


def _attn_kernel(x_ref, g_ref, bb_ref, wq_ref, wk_ref, wv_ref, wo_ref,
                 out_ref, qs_ref, ks_ref, vs_ref, xs_ref):
    B, S = x_ref.shape[0], x_ref.shape[1]
    nb = S // W
    scale = DH ** -0.5
    ii = lax.broadcasted_iota(jnp.int32, (nb, nb), 0)
    jj = lax.broadcasted_iota(jnp.int32, (nb, nb), 1)
    bi = lax.broadcasted_iota(jnp.int32, (nb, W, W), 0)
    wi = lax.broadcasted_iota(jnp.int32, (nb, W, W), 1)
    xi = lax.broadcasted_iota(jnp.int32, (nb, W, W), 2)
    amask = jnp.concatenate(
        [jnp.where(bi > 0, 0.0, NEG), jnp.where(xi <= wi, 0.0, NEG)], axis=2)

    def mm(a, w):
        return lax.dot_general(a, w, (((1,), (0,)), ((), ())),
                               preferred_element_type=jnp.float32)

    CH = 512
    nch = S // CH
    cb = CH // W

    for b in range(B):
        for c in range(nch):
            xc = x_ref[b, c * CH:(c + 1) * CH, :]
            m = jnp.mean(xc, axis=-1, keepdims=True)
            var = jnp.mean((xc - m) ** 2, axis=-1, keepdims=True)
            xs_ref[c * CH:(c + 1) * CH, :] = (
                (xc - m) / jnp.sqrt(var + 1e-5) * g_ref[0] + bb_ref[0]
            ).astype(jnp.bfloat16)
        for c in range(nch):
            xbc = xs_ref[c * CH:(c + 1) * CH, :]
            qc = (mm(xbc, wq_ref[...]) * scale).astype(jnp.bfloat16)
            kc = mm(xbc, wk_ref[...]).astype(jnp.bfloat16)
            vc = mm(xbc, wv_ref[...]).astype(jnp.bfloat16)
            for h in range(H):
                sl = slice(h * DH, (h + 1) * DH)
                bs = slice(c * cb, (c + 1) * cb)
                qs_ref[h, bs] = qc[:, sl].reshape(cb, W, DH)
                ks_ref[h, bs] = kc[:, sl].reshape(cb, W, DH)
                vs_ref[h, bs] = vc[:, sl].reshape(cb, W, DH)

        def head_body(h, carry):
            qb = qs_ref[h]
            kb = ks_ref[h]
            vb = vs_ref[h]
            qm = jnp.mean(qb.astype(jnp.float32), axis=1)
            km = jnp.mean(kb.astype(jnp.float32), axis=1)
            R = lax.dot_general(qm, km, (((1,), (1,)), ((), ())),
                                preferred_element_type=jnp.float32)
            Rm = jnp.where(jj < ii, R, NEG)
            Re = jnp.exp(Rm - jnp.max(Rm, axis=-1, keepdims=True))
            Rs = Re * (1.0 / jnp.sum(Re, axis=-1, keepdims=True))
            Rs = jnp.where(ii > 0, Rs, 0.0).astype(jnp.bfloat16)
            kr = lax.dot_general(Rs, kb, (((1,), (0,)), ((), ())),
                                 preferred_element_type=jnp.float32
                                 ).astype(jnp.bfloat16)
            vr = lax.dot_general(Rs, vb, (((1,), (0,)), ((), ())),
                                 preferred_element_type=jnp.float32
                                 ).astype(jnp.bfloat16)
            kcat = jnp.concatenate([kr, kb], axis=1)
            vcat = jnp.concatenate([vr, vb], axis=1)
            dots = lax.dot_general(qb, kcat, (((2,), (2,)), ((0,), (0,))),
                                   preferred_element_type=jnp.float32)
            dots = dots + amask
            mx = jnp.max(dots, axis=-1, keepdims=True)
            e = jnp.exp(dots - mx)
            rden = 1.0 / jnp.sum(e, axis=-1, keepdims=True)
            attn = (e * rden).astype(jnp.bfloat16)
            o = lax.dot_general(attn, vcat, (((2,), (1,)), ((0,), (0,))),
                                preferred_element_type=jnp.float32)
            qs_ref[h] = o.astype(jnp.bfloat16)
            return carry

        lax.fori_loop(0, H, head_body, 0)
        for c in range(nch):
            rs = slice(c * CH, (c + 1) * CH)
            bs = slice(c * cb, (c + 1) * cb)
            oc = jnp.concatenate(
                [qs_ref[h, bs].reshape(CH, DH) for h in range(H)], axis=1)
            out_ref[b, rs, :] = x_ref[b, rs, :] + mm(oc, wo_ref[...])


def _ffn_kernel(x_ref, g_ref, bb_ref, w1_ref, b1_ref, w2_ref, b2_ref,
                out_ref):
    x = x_ref[0]
    m = jnp.mean(x, axis=-1, keepdims=True)
    var = jnp.mean((x - m) ** 2, axis=-1, keepdims=True)
    hn = ((x - m) / jnp.sqrt(var + 1e-5) * g_ref[0] + bb_ref[0]
          ).astype(jnp.bfloat16)
    a = (lax.dot_general(hn, w1_ref[...], (((1,), (0,)), ((), ())),
                         preferred_element_type=jnp.float32) + b1_ref[0]
         ).astype(jnp.bfloat16)
    gl = jax.nn.gelu(a)
    out = lax.dot_general(gl, w2_ref[...], (((1,), (0,)), ((), ())),
                          preferred_element_type=jnp.float32) + b2_ref[0]
    out_ref[0] = x + out


def _head_kernel(x_ref, hT_ref, out_ref):
    xb = x_ref[0].astype(jnp.bfloat16)
    out_ref[0] = lax.dot_general(xb, hT_ref[...], (((1,), (0,)), ((), ())),
                                 preferred_element_type=jnp.float32)


def _pad_table(t, rows):
    return jnp.pad(t, ((0, rows - t.shape[0]), (0, 0)))


def kernel(value, depth, pos, params):
    B, S = value.shape
    value = value.astype(jnp.int32)
    depth = depth.astype(jnp.int32)
    pos = pos.astype(jnp.int32)
    z = jnp.zeros((B, 1), jnp.int32)
    vs = jnp.concatenate([z, value[:, :-1]], axis=1)
    ds = jnp.concatenate([z, depth[:, :-1]], axis=1)
    ps = jnp.concatenate([jnp.zeros((SD, B, 1), jnp.int32), pos[:, :, :-1]],
                         axis=2)
    tok = _pad_table(params["tok"], TPAD)
    dep = _pad_table(params["dep"], TPAD)
    spa = jnp.stack([_pad_table(params["spa"][a], TPAD) for a in range(SD)])
    sos = params["sos"].reshape(1, E)

    x = pl.pallas_call(
        _embed_kernel,
        out_shape=jax.ShapeDtypeStruct((B, S, E), jnp.float32),
    )(vs, ds, ps, tok, dep, spa, sos)

    bf = jnp.bfloat16
    for lp in params["layers"]:
        x = pl.pallas_call(
            _attn_kernel,
            out_shape=jax.ShapeDtypeStruct((B, S, E), jnp.float32),
            compiler_params=pltpu.CompilerParams(
                vmem_limit_bytes=62 * 1024 * 1024),
            scratch_shapes=[
                pltpu.VMEM((H, S // W, W, DH), jnp.bfloat16),
                pltpu.VMEM((H, S // W, W, DH), jnp.bfloat16),
                pltpu.VMEM((H, S // W, W, DH), jnp.bfloat16),
                pltpu.VMEM((S, E), jnp.bfloat16),
            ],
        )(x, lp["g1"].reshape(1, E), lp["bn1"].reshape(1, E),
          lp["wq"].astype(bf), lp["wk"].astype(bf), lp["wv"].astype(bf),
          lp["wo"].astype(bf))

        FCH = 256
        x = pl.pallas_call(
            _ffn_kernel,
            grid=(B, S // FCH),
            in_specs=[
                pl.BlockSpec((1, FCH, E), lambda b, c: (b, c, 0)),
                pl.BlockSpec((1, E), lambda b, c: (0, 0)),
                pl.BlockSpec((1, E), lambda b, c: (0, 0)),
                pl.BlockSpec((E, 4 * E), lambda b, c: (0, 0)),
                pl.BlockSpec((1, 4 * E), lambda b, c: (0, 0)),
                pl.BlockSpec((4 * E, E), lambda b, c: (0, 0)),
                pl.BlockSpec((1, E), lambda b, c: (0, 0)),
            ],
            out_specs=pl.BlockSpec((1, FCH, E), lambda b, c: (b, c, 0)),
            out_shape=jax.ShapeDtypeStruct((B, S, E), jnp.float32),
        )(x, lp["g2"].reshape(1, E), lp["bn2"].reshape(1, E),
          lp["w1"].astype(bf), lp["b1"].reshape(1, 4 * E),
          lp["w2"].astype(bf), lp["b2"].reshape(1, E))

    hT = jnp.pad(params["head"], ((0, TPAD - (VOCAB + 1)), (0, 0))
                 ).T.astype(bf)
    logits = pl.pallas_call(
        _head_kernel,
        grid=(B,),
        in_specs=[
            pl.BlockSpec((1, S, E), lambda b: (b, 0, 0)),
            pl.BlockSpec((E, TPAD), lambda b: (0, 0)),
        ],
        out_specs=pl.BlockSpec((1, S, TPAD), lambda b: (b, 0, 0)),
        out_shape=jax.ShapeDtypeStruct((B, S, TPAD), jnp.float32),
    )(x, hT)
    return logits[..., :VOCAB + 1]


# speedup vs baseline: 2.5012x; 1.0095x over previous
import jax
import jax.numpy as jnp
from jax import lax
from jax.experimental import pallas as pl
from jax.experimental.pallas import tpu as pltpu

E = 768
H = 12
DH = 64
W = 64
VOCAB = 16
SD = 3
NEG = -1e9
TPAD = 128


def _embed_kernel(vs_ref, ds_ref, ps_ref, tok_ref, dep_ref, spa_ref, sos_ref,
                  out_ref):
    B, S = vs_ref.shape
    CH = 512

    def gather(idx_row, tab):
        iot = lax.broadcasted_iota(jnp.int32, (TPAD, CH), 0)
        oh = jnp.where(idx_row == iot, 1.0, 0.0)
        return lax.dot_general(oh, tab, (((0,), (0,)), ((), ())),
                               preferred_element_type=jnp.float32)

    for b in range(B):
        for c in range(S // CH):
            r0 = c * CH
            acc = gather(vs_ref[b:b + 1, r0:r0 + CH], tok_ref[...])
            acc += gather(ds_ref[b:b + 1, r0:r0 + CH], dep_ref[...])
            for a in range(SD):
                acc += gather(ps_ref[a, b:b + 1, r0:r0 + CH], spa_ref[a])
            if c == 0:
                rid = lax.broadcasted_iota(jnp.int32, (CH, E), 0)
                acc = jnp.where(rid == 0, sos_ref[...], acc)
            out_ref[b, r0:r0 + CH, :] = acc


def _attn_kernel(x_ref, g_ref, bb_ref, wq_ref, wk_ref, wv_ref, wo_ref,
                 out_ref, qs_ref, ks_ref, vs_ref, xs_ref):
    B, S = x_ref.shape[0], x_ref.shape[1]
    nb = S // W
    scale = DH ** -0.5
    ii = lax.broadcasted_iota(jnp.int32, (nb, nb), 0)
    jj = lax.broadcasted_iota(jnp.int32, (nb, nb), 1)
    bi = lax.broadcasted_iota(jnp.int32, (nb, W, W), 0)
    wi = lax.broadcasted_iota(jnp.int32, (nb, W, W), 1)
    xi = lax.broadcasted_iota(jnp.int32, (nb, W, W), 2)
    amask = jnp.concatenate(
        [jnp.where(bi > 0, 0.0, NEG), jnp.where(xi <= wi, 0.0, NEG)], axis=2)

    def mm(a, w):
        return lax.dot_general(a, w, (((1,), (0,)), ((), ())),
                               preferred_element_type=jnp.float32)

    CH = 512
    nch = S // CH
    cb = CH // W

    for b in range(B):
        for c in range(nch):
            xc = x_ref[b, c * CH:(c + 1) * CH, :]
            m = jnp.mean(xc, axis=-1, keepdims=True)
            var = jnp.mean((xc - m) ** 2, axis=-1, keepdims=True)
            xs_ref[c * CH:(c + 1) * CH, :] = (
                (xc - m) / jnp.sqrt(var + 1e-5) * g_ref[0] + bb_ref[0]
            ).astype(jnp.bfloat16)
        for c in range(nch):
            xbc = xs_ref[c * CH:(c + 1) * CH, :]
            qc = (mm(xbc, wq_ref[...]) * scale).astype(jnp.bfloat16)
            kc = mm(xbc, wk_ref[...]).astype(jnp.bfloat16)
            vc = mm(xbc, wv_ref[...]).astype(jnp.bfloat16)
            for h in range(H):
                sl = slice(h * DH, (h + 1) * DH)
                bs = slice(c * cb, (c + 1) * cb)
                qs_ref[h, bs] = qc[:, sl].reshape(cb, W, DH)
                ks_ref[h, bs] = kc[:, sl].reshape(cb, W, DH)
                vs_ref[h, bs] = vc[:, sl].reshape(cb, W, DH)

        def one_head(h):
            qb = qs_ref[h]
            kb = ks_ref[h]
            vb = vs_ref[h]
            qm = jnp.mean(qb.astype(jnp.float32), axis=1)
            km = jnp.mean(kb.astype(jnp.float32), axis=1)
            R = lax.dot_general(qm, km, (((1,), (1,)), ((), ())),
                                preferred_element_type=jnp.float32)
            Rm = jnp.where(jj < ii, R, NEG)
            Re = jnp.exp(Rm - jnp.max(Rm, axis=-1, keepdims=True))
            Rs = Re * (1.0 / jnp.sum(Re, axis=-1, keepdims=True))
            Rs = jnp.where(ii > 0, Rs, 0.0).astype(jnp.bfloat16)
            kr = lax.dot_general(Rs, kb, (((1,), (0,)), ((), ())),
                                 preferred_element_type=jnp.float32
                                 ).astype(jnp.bfloat16)
            vr = lax.dot_general(Rs, vb, (((1,), (0,)), ((), ())),
                                 preferred_element_type=jnp.float32
                                 ).astype(jnp.bfloat16)
            kcat = jnp.concatenate([kr, kb], axis=1)
            vcat = jnp.concatenate([vr, vb], axis=1)
            dots = lax.dot_general(qb, kcat, (((2,), (2,)), ((0,), (0,))),
                                   preferred_element_type=jnp.float32)
            dots = dots + amask
            mx = jnp.max(dots, axis=-1, keepdims=True)
            e = jnp.exp(dots - mx)
            rden = 1.0 / jnp.sum(e, axis=-1, keepdims=True)
            attn = (e * rden).astype(jnp.bfloat16)
            o = lax.dot_general(attn, vcat, (((2,), (1,)), ((0,), (0,))),
                                preferred_element_type=jnp.float32)
            qs_ref[h] = o.astype(jnp.bfloat16)

        def head_body(hh, carry):
            one_head(hh * 2)
            one_head(hh * 2 + 1)
            return carry

        lax.fori_loop(0, H // 2, head_body, 0)
        for c in range(nch):
            rs = slice(c * CH, (c + 1) * CH)
            bs = slice(c * cb, (c + 1) * cb)
            oc = jnp.concatenate(
                [qs_ref[h, bs].reshape(CH, DH) for h in range(H)], axis=1)
            out_ref[b, rs, :] = x_ref[b, rs, :] + mm(oc, wo_ref[...])


def _ffn_kernel(x_ref, g_ref, bb_ref, w1_ref, b1_ref, w2_ref, b2_ref,
                out_ref):
    x = x_ref[0]
    m = jnp.mean(x, axis=-1, keepdims=True)
    var = jnp.mean((x - m) ** 2, axis=-1, keepdims=True)
    hn = ((x - m) / jnp.sqrt(var + 1e-5) * g_ref[0] + bb_ref[0]
          ).astype(jnp.bfloat16)
    a = (lax.dot_general(hn, w1_ref[...], (((1,), (0,)), ((), ())),
                         preferred_element_type=jnp.float32) + b1_ref[0]
         ).astype(jnp.bfloat16)
    gl = jax.nn.gelu(a)
    out = lax.dot_general(gl, w2_ref[...], (((1,), (0,)), ((), ())),
                          preferred_element_type=jnp.float32) + b2_ref[0]
    out_ref[0] = x + out


def _head_kernel(x_ref, hT_ref, out_ref):
    xb = x_ref[0].astype(jnp.bfloat16)
    out_ref[0] = lax.dot_general(xb, hT_ref[...], (((1,), (0,)), ((), ())),
                                 preferred_element_type=jnp.float32)


def _pad_table(t, rows):
    return jnp.pad(t, ((0, rows - t.shape[0]), (0, 0)))


def kernel(value, depth, pos, params):
    B, S = value.shape
    value = value.astype(jnp.int32)
    depth = depth.astype(jnp.int32)
    pos = pos.astype(jnp.int32)
    z = jnp.zeros((B, 1), jnp.int32)
    vs = jnp.concatenate([z, value[:, :-1]], axis=1)
    ds = jnp.concatenate([z, depth[:, :-1]], axis=1)
    ps = jnp.concatenate([jnp.zeros((SD, B, 1), jnp.int32), pos[:, :, :-1]],
                         axis=2)
    tok = _pad_table(params["tok"], TPAD)
    dep = _pad_table(params["dep"], TPAD)
    spa = jnp.stack([_pad_table(params["spa"][a], TPAD) for a in range(SD)])
    sos = params["sos"].reshape(1, E)

    x = pl.pallas_call(
        _embed_kernel,
        out_shape=jax.ShapeDtypeStruct((B, S, E), jnp.float32),
    )(vs, ds, ps, tok, dep, spa, sos)

    bf = jnp.bfloat16
    for lp in params["layers"]:
        x = pl.pallas_call(
            _attn_kernel,
            out_shape=jax.ShapeDtypeStruct((B, S, E), jnp.float32),
            compiler_params=pltpu.CompilerParams(
                vmem_limit_bytes=62 * 1024 * 1024),
            scratch_shapes=[
                pltpu.VMEM((H, S // W, W, DH), jnp.bfloat16),
                pltpu.VMEM((H, S // W, W, DH), jnp.bfloat16),
                pltpu.VMEM((H, S // W, W, DH), jnp.bfloat16),
                pltpu.VMEM((S, E), jnp.bfloat16),
            ],
        )(x, lp["g1"].reshape(1, E), lp["bn1"].reshape(1, E),
          lp["wq"].astype(bf), lp["wk"].astype(bf), lp["wv"].astype(bf),
          lp["wo"].astype(bf))

        FCH = 256
        x = pl.pallas_call(
            _ffn_kernel,
            grid=(B, S // FCH),
            in_specs=[
                pl.BlockSpec((1, FCH, E), lambda b, c: (b, c, 0)),
                pl.BlockSpec((1, E), lambda b, c: (0, 0)),
                pl.BlockSpec((1, E), lambda b, c: (0, 0)),
                pl.BlockSpec((E, 4 * E), lambda b, c: (0, 0)),
                pl.BlockSpec((1, 4 * E), lambda b, c: (0, 0)),
                pl.BlockSpec((4 * E, E), lambda b, c: (0, 0)),
                pl.BlockSpec((1, E), lambda b, c: (0, 0)),
            ],
            out_specs=pl.BlockSpec((1, FCH, E), lambda b, c: (b, c, 0)),
            out_shape=jax.ShapeDtypeStruct((B, S, E), jnp.float32),
        )(x, lp["g2"].reshape(1, E), lp["bn2"].reshape(1, E),
          lp["w1"].astype(bf), lp["b1"].reshape(1, 4 * E),
          lp["w2"].astype(bf), lp["b2"].reshape(1, E))

    hT = jnp.pad(params["head"], ((0, TPAD - (VOCAB + 1)), (0, 0))
                 ).T.astype(bf)
    logits = pl.pallas_call(
        _head_kernel,
        grid=(B,),
        in_specs=[
            pl.BlockSpec((1, S, E), lambda b: (b, 0, 0)),
            pl.BlockSpec((E, TPAD), lambda b: (0, 0)),
        ],
        out_specs=pl.BlockSpec((1, S, TPAD), lambda b: (b, 0, 0)),
        out_shape=jax.ShapeDtypeStruct((B, S, TPAD), jnp.float32),
    )(x, hT)
    return logits[..., :VOCAB + 1]
